# Initial kernel scaffold; baseline (speedup 1.0000x reference)
import jax
import jax.numpy as jnp
from jax.experimental import pallas as pl


def kernel(x):
    raise NotImplementedError("write your pallas kernel here")


# speedup vs baseline: 1.5490x; 1.5490x over previous
import functools

import jax
import jax.numpy as jnp
from jax import lax
from jax.experimental import pallas as pl
from jax.experimental.pallas import tpu as pltpu
from jax.experimental.pallas import tpu_sc as plsc

NUM_ROWS = 4096 * 26
NUM_COLS = 1000
NC = 2
NS = 16
NW = NC * NS
ROWS_PER_W = NUM_ROWS // NW
LANES = 16
CHUNK = LANES
NCHUNKS = ROWS_PER_W // CHUNK
NBUF = 4
BUF_WORDS = CHUNK * NUM_COLS

_mesh = plsc.VectorSubcoreMesh(core_axis_name="c", subcore_axis_name="s")


@functools.partial(
    pl.kernel,
    out_type=jax.ShapeDtypeStruct((NUM_ROWS * NUM_COLS,), jnp.float32),
    mesh=_mesh,
    scratch_types=(
        [pltpu.VMEM((ROWS_PER_W,), jnp.int32)]
        + [pltpu.VMEM((BUF_WORDS,), jnp.float32) for _ in range(NBUF)]
        + [pltpu.SemaphoreType.DMA for _ in range(NBUF)]
    ),
    compiler_params=pltpu.CompilerParams(needs_layout_passes=False),
)
def _one_hot_sc(x_hbm, out_hbm, idx_v, b0, b1, b2, b3, s0, s1, s2, s3):
    bufs = [b0, b1, b2, b3]
    sems = [s0, s1, s2, s3]
    wid = lax.axis_index("s") * NC + lax.axis_index("c")
    base_row = wid * ROWS_PER_W

    pltpu.sync_copy(x_hbm.at[pl.ds(base_row, ROWS_PER_W)], idx_v)

    zeros16 = jnp.zeros((LANES,), jnp.float32)
    ones16 = jnp.ones((LANES,), jnp.float32)
    row_off = lax.iota(jnp.int32, 16) * NUM_COLS

    def _zero(i, carry):
        for b in range(NBUF):
            bufs[b][pl.ds(i * LANES, LANES)] = zeros16
        return carry

    lax.fori_loop(0, BUF_WORDS // LANES, _zero, 0)

    def scatter_ones(b, c):
        idx = idx_v[pl.ds(c * CHUNK, CHUNK)]
        plsc.store_scatter(bufs[b], [row_off + idx], ones16)

    def dma(b, c):
        dst = out_hbm.at[pl.ds((base_row + c * CHUNK) * NUM_COLS, BUF_WORDS)]
        return pltpu.make_async_copy(bufs[b], dst, sems[b])

    for b in range(NBUF):
        scatter_ones(b, b)
        dma(b, b).start()

    def step(g, carry):
        for b in range(NBUF):
            c = g * NBUF + b
            dma(b, c).wait()
            old_idx = idx_v[pl.ds((c - NBUF) * CHUNK, CHUNK)]
            plsc.store_scatter(bufs[b], [row_off + old_idx], zeros16)
            scatter_ones(b, c)
            dma(b, c).start()
        return carry

    lax.fori_loop(1, NCHUNKS // NBUF, step, 0)

    for b in range(NBUF):
        dma(b, 0).wait()


def kernel(x):
    x = x.reshape(-1).astype(jnp.int32)
    out = _one_hot_sc(x)
    return out.reshape(4096, 26, NUM_COLS)
